# baseline (speedup 1.0000x reference)
import functools

import jax
import jax.numpy as jnp
from jax import lax
from jax.experimental import pallas as pl
from jax.experimental.pallas import tpu as pltpu
from jax.experimental.pallas import tpu_sc as plsc

_NUM_STEP = 3
_NEG = -1e30
_EPS = 1e-10
_BATCH = 128
_LANES = 16
_SUBCORES = 16


def _lstm_scores_body(x_ref, n2g_ref, h_in, c_in, qs_in, wih_ref, whh_ref,
                      bias_ref, h_out, c_out, p_out, *, nblk, dim):
    j = pl.program_id(0)

    @pl.when(j == 0)
    def lstm_stage():
        gates = (
            lax.dot_general(qs_in[...], wih_ref[...], (((1,), (0,)), ((), ())),
                            preferred_element_type=jnp.float32)
            + lax.dot_general(h_in[...], whh_ref[...], (((1,), (0,)), ((), ())),
                              preferred_element_type=jnp.float32)
            + bias_ref[...]
        )
        i = jax.nn.sigmoid(gates[:, 0 * dim:1 * dim])
        f = jax.nn.sigmoid(gates[:, 1 * dim:2 * dim])
        g = jnp.tanh(gates[:, 2 * dim:3 * dim])
        o = jax.nn.sigmoid(gates[:, 3 * dim:4 * dim])
        c_new = f * c_in[...] + i * g
        c_out[...] = c_new
        h_out[...] = o * jnp.tanh(c_new)

    @pl.when(j > 0)
    def score_stage():
        x_blk = x_ref[...]
        n2g = n2g_ref[0, 0, :]
        h = h_out[...].astype(jnp.bfloat16)
        scores_t = lax.dot_general(h, x_blk, (((1,), (1,)), ((), ())),
                                   preferred_element_type=jnp.float32)
        gid = lax.broadcasted_iota(jnp.int32, scores_t.shape, 0)
        member = gid == n2g[None, :]
        masked = jnp.where(member, scores_t, 0.0)
        ones = jnp.ones((1, masked.shape[0]), jnp.float32)
        p_out[...] = lax.dot_general(ones, masked, (((1,), (0,)), ((), ())),
                                     preferred_element_type=jnp.float32
                                     ).reshape(1, 1, -1)


def _sc_softmax_body(p_hbm, g_hbm, a_hbm, norm_hbm, p_v, g_v, e_v, m_v,
                     s_v, m2_v, s2_v, stage_v, sh_m, sh_s, *,
                     chunk, last_chunk, groups, batch):
    wid = lax.axis_index("s")
    base = wid * chunk
    lane_off = lax.iota(jnp.int32, _LANES) * batch
    tail = chunk - last_chunk

    @pl.when(wid < _SUBCORES - 1)
    def full_load():
        pltpu.sync_copy(p_hbm.at[pl.ds(base, chunk)], p_v)
        pltpu.sync_copy(g_hbm.at[pl.ds(base, chunk)], g_v)

    @pl.when(wid == _SUBCORES - 1)
    def tail_load():
        pltpu.sync_copy(p_hbm.at[pl.ds(base, last_chunk)],
                        p_v.at[pl.ds(0, last_chunk)])
        pltpu.sync_copy(g_hbm.at[pl.ds(base, last_chunk)],
                        g_v.at[pl.ds(0, last_chunk)])
        for i in range(tail // _LANES):
            sl = pl.ds(last_chunk + i * _LANES, _LANES)
            p_v[sl] = jnp.full((_LANES,), _NEG, jnp.float32)
            g_v[sl] = jnp.full((_LANES,), batch - 1, jnp.int32)

    for i in range(2 * batch):
        m2_v[pl.ds(i * _LANES, _LANES)] = jnp.full((_LANES,), _NEG,
                                                   jnp.float32)
        s2_v[pl.ds(i * _LANES, _LANES)] = jnp.zeros((_LANES,), jnp.float32)
    set1 = _LANES * batch

    def max_group(i, carry):
        sl0 = pl.ds((2 * i) * _LANES, _LANES)
        sl1 = pl.ds((2 * i + 1) * _LANES, _LANES)
        idx0 = lane_off + g_v[sl0]
        idx1 = set1 + lane_off + g_v[sl1]
        old0 = plsc.load_gather(m2_v, [idx0])
        old1 = plsc.load_gather(m2_v, [idx1])
        plsc.store_scatter(m2_v, [idx0], jnp.maximum(old0, p_v[sl0]))
        plsc.store_scatter(m2_v, [idx1], jnp.maximum(old1, p_v[sl1]))
        return carry

    lax.fori_loop(0, groups // 2, max_group, 0)

    for c in range(batch // _LANES):
        acc = jnp.full((_LANES,), _NEG, jnp.float32)
        for s in range(2 * _LANES):
            acc = jnp.maximum(acc, m2_v[pl.ds(s * batch + c * _LANES, _LANES)])
        m_v[pl.ds(c * _LANES, _LANES)] = acc
    pltpu.sync_copy(m_v, sh_m.at[wid])
    plsc.subcore_barrier()
    pltpu.sync_copy(sh_m, stage_v)
    for c in range(batch // _LANES):
        acc = jnp.full((_LANES,), _NEG, jnp.float32)
        for r in range(_SUBCORES):
            acc = jnp.maximum(acc, stage_v[r, pl.ds(c * _LANES, _LANES)])
        m_v[pl.ds(c * _LANES, _LANES)] = acc

    def sum_group(i, carry):
        sl0 = pl.ds((2 * i) * _LANES, _LANES)
        sl1 = pl.ds((2 * i + 1) * _LANES, _LANES)
        gv0 = g_v[sl0]
        gv1 = g_v[sl1]
        ev0 = jnp.exp(p_v[sl0] - plsc.load_gather(m_v, [gv0]))
        ev1 = jnp.exp(p_v[sl1] - plsc.load_gather(m_v, [gv1]))
        e_v[sl0] = ev0
        e_v[sl1] = ev1
        idx0 = lane_off + gv0
        idx1 = set1 + lane_off + gv1
        old0 = plsc.load_gather(s2_v, [idx0])
        old1 = plsc.load_gather(s2_v, [idx1])
        plsc.store_scatter(s2_v, [idx0], old0 + ev0)
        plsc.store_scatter(s2_v, [idx1], old1 + ev1)
        return carry

    lax.fori_loop(0, groups // 2, sum_group, 0)

    for c in range(batch // _LANES):
        acc = jnp.zeros((_LANES,), jnp.float32)
        for s in range(2 * _LANES):
            acc = acc + s2_v[pl.ds(s * batch + c * _LANES, _LANES)]
        s_v[pl.ds(c * _LANES, _LANES)] = acc
    pltpu.sync_copy(s_v, sh_s.at[wid])
    plsc.subcore_barrier()
    pltpu.sync_copy(sh_s, stage_v)
    for c in range(batch // _LANES):
        acc = jnp.zeros((_LANES,), jnp.float32)
        for r in range(_SUBCORES):
            acc = acc + stage_v[r, pl.ds(c * _LANES, _LANES)]
        s_v[pl.ds(c * _LANES, _LANES)] = acc + _EPS

    @pl.when(wid == 0)
    def norm_store():
        pltpu.sync_copy(s_v, norm_hbm)

    @pl.when(wid < _SUBCORES - 1)
    def full_store():
        pltpu.sync_copy(e_v, a_hbm.at[pl.ds(base, chunk)])

    @pl.when(wid == _SUBCORES - 1)
    def tail_store():
        pltpu.sync_copy(e_v.at[pl.ds(0, last_chunk)],
                        a_hbm.at[pl.ds(base, last_chunk)])


def _pool_lstm_scores_body(x_ref, n2g_ref, a_ref, norm_ref, h_in, c_in,
                           wih_ref, whh_ref, bias_ref, h_out, c_out, p_out,
                           o_acc, *, nblk, dim):
    j = pl.program_id(0)

    @pl.when(j == 0)
    def init():
        o_acc[...] = jnp.zeros_like(o_acc)

    @pl.when(j < nblk)
    def pool_stage():
        x_blk = x_ref[...]
        n2g = n2g_ref[0, 0, :]
        a_blk = a_ref[0, :, :].astype(jnp.bfloat16)
        gid = lax.broadcasted_iota(jnp.int16,
                                   (o_acc.shape[0], x_blk.shape[0]), 0)
        n2gb = n2g.astype(jnp.int16)
        amask = jnp.where(gid == n2gb[None, :], a_blk, jnp.bfloat16(0))
        o_acc[...] += lax.dot_general(amask, x_blk, (((1,), (0,)), ((), ())),
                                      preferred_element_type=jnp.float32)

    @pl.when(j == nblk)
    def lstm_stage():
        pooled = o_acc[...] / norm_ref[...].reshape(o_acc.shape[0], 1)
        gates = (
            lax.dot_general(h_in[...], wih_ref[0:dim, :],
                            (((1,), (0,)), ((), ())),
                            preferred_element_type=jnp.float32)
            + lax.dot_general(pooled, wih_ref[dim:2 * dim, :],
                              (((1,), (0,)), ((), ())),
                              preferred_element_type=jnp.float32)
            + lax.dot_general(h_in[...], whh_ref[...],
                              (((1,), (0,)), ((), ())),
                              preferred_element_type=jnp.float32)
            + bias_ref[...]
        )
        i = jax.nn.sigmoid(gates[:, 0 * dim:1 * dim])
        f = jax.nn.sigmoid(gates[:, 1 * dim:2 * dim])
        g = jnp.tanh(gates[:, 2 * dim:3 * dim])
        o = jax.nn.sigmoid(gates[:, 3 * dim:4 * dim])
        c_new = f * c_in[...] + i * g
        c_out[...] = c_new
        h_out[...] = o * jnp.tanh(c_new)

    @pl.when(j > nblk)
    def score_stage():
        x_blk = x_ref[...]
        n2g = n2g_ref[0, 0, :]
        h = h_out[...].astype(jnp.bfloat16)
        scores_t = lax.dot_general(h, x_blk, (((1,), (1,)), ((), ())),
                                   preferred_element_type=jnp.float32)
        gid = lax.broadcasted_iota(jnp.int32, scores_t.shape, 0)
        masked = jnp.where(gid == n2g[None, :], scores_t, 0.0)
        ones = jnp.ones((1, masked.shape[0]), jnp.float32)
        p_out[...] = lax.dot_general(ones, masked, (((1,), (0,)), ((), ())),
                                     preferred_element_type=jnp.float32
                                     ).reshape(1, 1, -1)


def _pool_body(x_ref, n2g_ref, a_ref, norm_ref, h_in, qs_out, o_acc, *,
               nblk, dim):
    j = pl.program_id(0)

    @pl.when(j == 0)
    def init():
        o_acc[...] = jnp.zeros_like(o_acc)

    x_blk = x_ref[...]
    n2g = n2g_ref[0, 0, :]
    a_blk = a_ref[0, :, :].astype(jnp.bfloat16)
    gid = lax.broadcasted_iota(jnp.int16,
                               (o_acc.shape[0], x_blk.shape[0]), 0)
    amask = jnp.where(gid == n2g.astype(jnp.int16)[None, :], a_blk,
                      jnp.bfloat16(0))
    o_acc[...] += lax.dot_general(amask, x_blk, (((1,), (0,)), ((), ())),
                                  preferred_element_type=jnp.float32)

    @pl.when(j == nblk - 1)
    def finish():
        qs_out[:, 0:dim] = h_in[...]
        qs_out[:, dim:2 * dim] = (o_acc[...]
                                  / norm_ref[...].reshape(o_acc.shape[0], 1))


def kernel(input, node2graph, batch_size, W_ih, W_hh, b_ih, b_hh):
    x = input
    n, dim = x.shape
    batch = _BATCH

    nb = 25600
    npd = -(-n // nb) * nb
    nblk = npd // nb

    chunk = -(-n // (_SUBCORES * _LANES)) * _LANES
    last_chunk = n - (_SUBCORES - 1) * chunk
    assert 0 < last_chunk <= chunk and last_chunk % _LANES == 0
    assert chunk % _LANES == 0 and (chunk // _LANES) % 2 == 0
    groups = chunk // _LANES

    n2gp = jnp.pad(node2graph.astype(jnp.int32), (0, npd - n),
                   constant_values=-1)
    n2g3 = n2gp.reshape(nblk, 1, nb)
    wih_t = W_ih.T
    whh_t = W_hh.T
    bias = (b_ih + b_hh).reshape(1, 4 * dim)

    tc_scores = pl.pallas_call(
        functools.partial(_lstm_scores_body, nblk=nblk, dim=dim),
        grid=(nblk + 1,),
        in_specs=[
            pl.BlockSpec((nb, dim), lambda j: (jnp.maximum(j, 1) - 1, 0)),
            pl.BlockSpec((1, 1, nb), lambda j: (jnp.maximum(j, 1) - 1, 0, 0)),
            pl.BlockSpec((batch, dim), lambda j: (0, 0)),
            pl.BlockSpec((batch, dim), lambda j: (0, 0)),
            pl.BlockSpec((batch, 2 * dim), lambda j: (0, 0)),
            pl.BlockSpec((2 * dim, 4 * dim), lambda j: (0, 0)),
            pl.BlockSpec((dim, 4 * dim), lambda j: (0, 0)),
            pl.BlockSpec((1, 4 * dim), lambda j: (0, 0)),
        ],
        out_specs=[
            pl.BlockSpec((batch, dim), lambda j: (0, 0)),
            pl.BlockSpec((batch, dim), lambda j: (0, 0)),
            pl.BlockSpec((1, 1, nb), lambda j: (jnp.maximum(j, 1) - 1, 0, 0)),
        ],
        out_shape=[
            jax.ShapeDtypeStruct((batch, dim), jnp.float32),
            jax.ShapeDtypeStruct((batch, dim), jnp.float32),
            jax.ShapeDtypeStruct((nblk, 1, nb), jnp.float32),
        ],
        compiler_params=pltpu.CompilerParams(
            dimension_semantics=("arbitrary",),
        ),
    )

    mesh = plsc.VectorSubcoreMesh(core_axis_name="c", subcore_axis_name="s",
                                  num_cores=1, num_subcores=_SUBCORES)
    sc_softmax = pl.kernel(
        functools.partial(_sc_softmax_body, chunk=chunk,
                          last_chunk=last_chunk, groups=groups, batch=batch),
        out_type=[jax.ShapeDtypeStruct((npd,), jnp.float32),
                  jax.ShapeDtypeStruct((batch,), jnp.float32)],
        mesh=mesh,
        scratch_types=[
            pltpu.VMEM((chunk,), jnp.float32),
            pltpu.VMEM((chunk,), jnp.int32),
            pltpu.VMEM((chunk,), jnp.float32),
            pltpu.VMEM((batch,), jnp.float32),
            pltpu.VMEM((batch,), jnp.float32),
            pltpu.VMEM((2 * _LANES * batch,), jnp.float32),
            pltpu.VMEM((2 * _LANES * batch,), jnp.float32),
            pltpu.VMEM((_SUBCORES, batch), jnp.float32),
            pltpu.VMEM_SHARED((_SUBCORES, batch), jnp.float32),
            pltpu.VMEM_SHARED((_SUBCORES, batch), jnp.float32),
        ],
        compiler_params=pltpu.CompilerParams(needs_layout_passes=False),
    )

    tc_ba = pl.pallas_call(
        functools.partial(_pool_lstm_scores_body, nblk=nblk, dim=dim),
        grid=(2 * nblk + 1,),
        in_specs=[
            pl.BlockSpec((nb, dim), lambda j: (
                jnp.where(j < nblk, j, jnp.maximum(j - nblk - 1, 0)), 0)),
            pl.BlockSpec((1, 1, nb), lambda j: (
                jnp.where(j < nblk, j, jnp.maximum(j - nblk - 1, 0)), 0, 0)),
            pl.BlockSpec((1, 1, nb), lambda j: (jnp.minimum(j, nblk - 1),
                                                0, 0)),
            pl.BlockSpec((1, batch), lambda j: (0, 0)),
            pl.BlockSpec((batch, dim), lambda j: (0, 0)),
            pl.BlockSpec((batch, dim), lambda j: (0, 0)),
            pl.BlockSpec((2 * dim, 4 * dim), lambda j: (0, 0)),
            pl.BlockSpec((dim, 4 * dim), lambda j: (0, 0)),
            pl.BlockSpec((1, 4 * dim), lambda j: (0, 0)),
        ],
        out_specs=[
            pl.BlockSpec((batch, dim), lambda j: (0, 0)),
            pl.BlockSpec((batch, dim), lambda j: (0, 0)),
            pl.BlockSpec((1, 1, nb), lambda j: (jnp.maximum(j - nblk - 1, 0),
                                                0, 0)),
        ],
        out_shape=[
            jax.ShapeDtypeStruct((batch, dim), jnp.float32),
            jax.ShapeDtypeStruct((batch, dim), jnp.float32),
            jax.ShapeDtypeStruct((nblk, 1, nb), jnp.float32),
        ],
        scratch_shapes=[pltpu.VMEM((batch, dim), jnp.float32)],
        compiler_params=pltpu.CompilerParams(
            dimension_semantics=("arbitrary",),
        ),
    )

    tc_pool = pl.pallas_call(
        functools.partial(_pool_body, nblk=nblk, dim=dim),
        grid=(nblk,),
        in_specs=[
            pl.BlockSpec((nb, dim), lambda j: (j, 0)),
            pl.BlockSpec((1, 1, nb), lambda j: (j, 0, 0)),
            pl.BlockSpec((1, 1, nb), lambda j: (j, 0, 0)),
            pl.BlockSpec((1, batch), lambda j: (0, 0)),
            pl.BlockSpec((batch, dim), lambda j: (0, 0)),
        ],
        out_specs=pl.BlockSpec((batch, 2 * dim), lambda j: (0, 0)),
        out_shape=jax.ShapeDtypeStruct((batch, 2 * dim), jnp.float32),
        scratch_shapes=[pltpu.VMEM((batch, dim), jnp.float32)],
        compiler_params=pltpu.CompilerParams(
            dimension_semantics=("arbitrary",),
        ),
    )

    xb = jnp.pad(x.astype(jnp.bfloat16), ((0, npd - n), (0, 0)))
    h = jnp.zeros((batch, dim), jnp.float32)
    c = jnp.zeros((batch, dim), jnp.float32)
    qs0 = jnp.zeros((batch, 2 * dim), jnp.float32)
    h, c, p = tc_scores(xb, n2g3, h, c, qs0, wih_t, whh_t, bias)
    for _ in range(_NUM_STEP - 1):
        e, norm = sc_softmax(p.reshape(-1), n2gp)
        h, c, p = tc_ba(xb, n2g3, e.reshape(nblk, 1, nb),
                        norm.reshape(1, batch), h, c, wih_t, whh_t, bias)
    e, norm = sc_softmax(p.reshape(-1), n2gp)
    return tc_pool(xb, n2g3, e.reshape(nblk, 1, nb), norm.reshape(1, batch),
                   h)


# speedup vs baseline: 1.0153x; 1.0153x over previous
import functools

import jax
import jax.numpy as jnp
from jax import lax
from jax.experimental import pallas as pl
from jax.experimental.pallas import tpu as pltpu
from jax.experimental.pallas import tpu_sc as plsc

_NUM_STEP = 3
_NEG = -1e30
_EPS = 1e-10
_BATCH = 128
_LANES = 16
_SUBCORES = 16


def _lstm_scores_body(x_ref, n2g_ref, h_in, c_in, qs_in, wih_ref, whh_ref,
                      bias_ref, h_out, c_out, p_out, *, nblk, dim):
    j = pl.program_id(0)

    @pl.when(j == 0)
    def lstm_stage():
        gates = (
            lax.dot_general(qs_in[...], wih_ref[...], (((1,), (0,)), ((), ())),
                            preferred_element_type=jnp.float32)
            + lax.dot_general(h_in[...], whh_ref[...], (((1,), (0,)), ((), ())),
                              preferred_element_type=jnp.float32)
            + bias_ref[...]
        )
        i = jax.nn.sigmoid(gates[:, 0 * dim:1 * dim])
        f = jax.nn.sigmoid(gates[:, 1 * dim:2 * dim])
        g = jnp.tanh(gates[:, 2 * dim:3 * dim])
        o = jax.nn.sigmoid(gates[:, 3 * dim:4 * dim])
        c_new = f * c_in[...] + i * g
        c_out[...] = c_new
        h_out[...] = o * jnp.tanh(c_new)

    @pl.when(j > 0)
    def score_stage():
        x_blk = x_ref[...]
        n2g = n2g_ref[0, 0, :]
        h = h_out[...].astype(jnp.bfloat16)
        scores_t = lax.dot_general(h, x_blk, (((1,), (1,)), ((), ())),
                                   preferred_element_type=jnp.float32)
        gid = lax.broadcasted_iota(jnp.int32, scores_t.shape, 0)
        member = gid == n2g[None, :]
        masked = jnp.where(member, scores_t, 0.0)
        ones = jnp.ones((1, masked.shape[0]), jnp.float32)
        p_out[...] = lax.dot_general(ones, masked, (((1,), (0,)), ((), ())),
                                     preferred_element_type=jnp.float32
                                     ).reshape(1, 1, -1)


def _sc_softmax_body(p_hbm, g_hbm, a_hbm, norm_hbm, p_v, g_v, e_v, m_v,
                     s_v, m2_v, s2_v, stage_v, sh_m, sh_s, *,
                     chunk, last_chunk, groups, batch):
    wid = lax.axis_index("s")
    base = wid * chunk
    lane_off = lax.iota(jnp.int32, _LANES) * batch
    tail = chunk - last_chunk

    @pl.when(wid < _SUBCORES - 1)
    def full_load():
        pltpu.sync_copy(p_hbm.at[pl.ds(base, chunk)], p_v)
        pltpu.sync_copy(g_hbm.at[pl.ds(base, chunk)], g_v)

    @pl.when(wid == _SUBCORES - 1)
    def tail_load():
        pltpu.sync_copy(p_hbm.at[pl.ds(base, last_chunk)],
                        p_v.at[pl.ds(0, last_chunk)])
        pltpu.sync_copy(g_hbm.at[pl.ds(base, last_chunk)],
                        g_v.at[pl.ds(0, last_chunk)])
        for i in range(tail // _LANES):
            sl = pl.ds(last_chunk + i * _LANES, _LANES)
            p_v[sl] = jnp.full((_LANES,), _NEG, jnp.float32)
            g_v[sl] = jnp.full((_LANES,), batch - 1, jnp.int32)

    for i in range(2 * batch):
        m2_v[pl.ds(i * _LANES, _LANES)] = jnp.full((_LANES,), _NEG,
                                                   jnp.float32)
        s2_v[pl.ds(i * _LANES, _LANES)] = jnp.zeros((_LANES,), jnp.float32)
    set1 = _LANES * batch

    def max_group(i, carry):
        sl0 = pl.ds((2 * i) * _LANES, _LANES)
        sl1 = pl.ds((2 * i + 1) * _LANES, _LANES)
        idx0 = lane_off + g_v[sl0]
        idx1 = set1 + lane_off + g_v[sl1]
        old0 = plsc.load_gather(m2_v, [idx0])
        old1 = plsc.load_gather(m2_v, [idx1])
        plsc.store_scatter(m2_v, [idx0], jnp.maximum(old0, p_v[sl0]))
        plsc.store_scatter(m2_v, [idx1], jnp.maximum(old1, p_v[sl1]))
        return carry

    lax.fori_loop(0, groups // 2, max_group, 0)

    for c in range(batch // _LANES):
        acc = jnp.full((_LANES,), _NEG, jnp.float32)
        for s in range(2 * _LANES):
            acc = jnp.maximum(acc, m2_v[pl.ds(s * batch + c * _LANES, _LANES)])
        m_v[pl.ds(c * _LANES, _LANES)] = acc
    pltpu.sync_copy(m_v, sh_m.at[wid])
    plsc.subcore_barrier()
    pltpu.sync_copy(sh_m, stage_v)
    for c in range(batch // _LANES):
        acc = jnp.full((_LANES,), _NEG, jnp.float32)
        for r in range(_SUBCORES):
            acc = jnp.maximum(acc, stage_v[r, pl.ds(c * _LANES, _LANES)])
        m_v[pl.ds(c * _LANES, _LANES)] = acc

    def sum_group(i, carry):
        sl0 = pl.ds((2 * i) * _LANES, _LANES)
        sl1 = pl.ds((2 * i + 1) * _LANES, _LANES)
        gv0 = g_v[sl0]
        gv1 = g_v[sl1]
        ev0 = jnp.exp(p_v[sl0] - plsc.load_gather(m_v, [gv0]))
        ev1 = jnp.exp(p_v[sl1] - plsc.load_gather(m_v, [gv1]))
        e_v[sl0] = ev0
        e_v[sl1] = ev1
        idx0 = lane_off + gv0
        idx1 = set1 + lane_off + gv1
        old0 = plsc.load_gather(s2_v, [idx0])
        old1 = plsc.load_gather(s2_v, [idx1])
        plsc.store_scatter(s2_v, [idx0], old0 + ev0)
        plsc.store_scatter(s2_v, [idx1], old1 + ev1)
        return carry

    lax.fori_loop(0, groups // 2, sum_group, 0)

    for c in range(batch // _LANES):
        acc = jnp.zeros((_LANES,), jnp.float32)
        for s in range(2 * _LANES):
            acc = acc + s2_v[pl.ds(s * batch + c * _LANES, _LANES)]
        s_v[pl.ds(c * _LANES, _LANES)] = acc
    pltpu.sync_copy(s_v, sh_s.at[wid])
    plsc.subcore_barrier()
    pltpu.sync_copy(sh_s, stage_v)
    for c in range(batch // _LANES):
        acc = jnp.zeros((_LANES,), jnp.float32)
        for r in range(_SUBCORES):
            acc = acc + stage_v[r, pl.ds(c * _LANES, _LANES)]
        s_v[pl.ds(c * _LANES, _LANES)] = acc + _EPS

    @pl.when(wid == 0)
    def norm_store():
        pltpu.sync_copy(s_v, norm_hbm)

    @pl.when(wid < _SUBCORES - 1)
    def full_store():
        pltpu.sync_copy(e_v, a_hbm.at[pl.ds(base, chunk)])

    @pl.when(wid == _SUBCORES - 1)
    def tail_store():
        pltpu.sync_copy(e_v.at[pl.ds(0, last_chunk)],
                        a_hbm.at[pl.ds(base, last_chunk)])


def _pool_lstm_scores_body(x_ref, n2g_ref, a_ref, norm_ref, h_in, c_in,
                           wih_ref, whh_ref, bias_ref, h_out, c_out, p_out,
                           o_acc, *, nblk, dim):
    j = pl.program_id(0)

    @pl.when(j == 0)
    def init():
        o_acc[...] = jnp.zeros_like(o_acc)

    @pl.when(j < nblk)
    def pool_stage():
        x_blk = x_ref[...]
        n2g = n2g_ref[0, 0, :]
        a_blk = a_ref[0, :, :].astype(jnp.bfloat16)
        gid = lax.broadcasted_iota(jnp.int16,
                                   (o_acc.shape[0], x_blk.shape[0]), 0)
        n2gb = n2g.astype(jnp.int16)
        amask = jnp.where(gid == n2gb[None, :], a_blk, jnp.bfloat16(0))
        o_acc[...] += lax.dot_general(amask, x_blk, (((1,), (0,)), ((), ())),
                                      preferred_element_type=jnp.float32)

    @pl.when(j == nblk)
    def lstm_stage():
        pooled = o_acc[...] / norm_ref[...].reshape(o_acc.shape[0], 1)
        gates = (
            lax.dot_general(h_in[...], wih_ref[0:dim, :],
                            (((1,), (0,)), ((), ())),
                            preferred_element_type=jnp.float32)
            + lax.dot_general(pooled, wih_ref[dim:2 * dim, :],
                              (((1,), (0,)), ((), ())),
                              preferred_element_type=jnp.float32)
            + lax.dot_general(h_in[...], whh_ref[...],
                              (((1,), (0,)), ((), ())),
                              preferred_element_type=jnp.float32)
            + bias_ref[...]
        )
        i = jax.nn.sigmoid(gates[:, 0 * dim:1 * dim])
        f = jax.nn.sigmoid(gates[:, 1 * dim:2 * dim])
        g = jnp.tanh(gates[:, 2 * dim:3 * dim])
        o = jax.nn.sigmoid(gates[:, 3 * dim:4 * dim])
        c_new = f * c_in[...] + i * g
        c_out[...] = c_new
        h_out[...] = o * jnp.tanh(c_new)

    @pl.when(j > nblk)
    def score_stage():
        x_blk = x_ref[...]
        n2g = n2g_ref[0, 0, :]
        h = h_out[...].astype(jnp.bfloat16)
        scores_t = lax.dot_general(h, x_blk, (((1,), (1,)), ((), ())),
                                   preferred_element_type=jnp.float32)
        gid = lax.broadcasted_iota(jnp.int32, scores_t.shape, 0)
        masked = jnp.where(gid == n2g[None, :], scores_t, 0.0)
        ones = jnp.ones((1, masked.shape[0]), jnp.float32)
        p_out[...] = lax.dot_general(ones, masked, (((1,), (0,)), ((), ())),
                                     preferred_element_type=jnp.float32
                                     ).reshape(1, 1, -1)


def _pool_body(x_ref, n2g_ref, a_ref, norm_ref, h_in, qs_out, o_acc, *,
               nblk, dim):
    j = pl.program_id(0)

    @pl.when(j == 0)
    def init():
        o_acc[...] = jnp.zeros_like(o_acc)

    x_blk = x_ref[...]
    n2g = n2g_ref[0, 0, :]
    a_blk = a_ref[0, :, :].astype(jnp.bfloat16)
    gid = lax.broadcasted_iota(jnp.int16,
                               (o_acc.shape[0], x_blk.shape[0]), 0)
    amask = jnp.where(gid == n2g.astype(jnp.int16)[None, :], a_blk,
                      jnp.bfloat16(0))
    o_acc[...] += lax.dot_general(amask, x_blk, (((1,), (0,)), ((), ())),
                                  preferred_element_type=jnp.float32)

    @pl.when(j == nblk - 1)
    def finish():
        qs_out[:, 0:dim] = h_in[...]
        qs_out[:, dim:2 * dim] = (o_acc[...]
                                  / norm_ref[...].reshape(o_acc.shape[0], 1))


def kernel(input, node2graph, batch_size, W_ih, W_hh, b_ih, b_hh):
    x = input
    n, dim = x.shape
    batch = _BATCH

    nb = 12800
    npd = -(-n // nb) * nb
    nblk = npd // nb

    chunk = -(-n // (_SUBCORES * _LANES)) * _LANES
    last_chunk = n - (_SUBCORES - 1) * chunk
    assert 0 < last_chunk <= chunk and last_chunk % _LANES == 0
    assert chunk % _LANES == 0 and (chunk // _LANES) % 2 == 0
    groups = chunk // _LANES

    n2gp = jnp.pad(node2graph.astype(jnp.int32), (0, npd - n),
                   constant_values=-1)
    n2g3 = n2gp.reshape(nblk, 1, nb)
    wih_t = W_ih.T
    whh_t = W_hh.T
    bias = (b_ih + b_hh).reshape(1, 4 * dim)

    tc_scores = pl.pallas_call(
        functools.partial(_lstm_scores_body, nblk=nblk, dim=dim),
        grid=(nblk + 1,),
        in_specs=[
            pl.BlockSpec((nb, dim), lambda j: (jnp.maximum(j, 1) - 1, 0)),
            pl.BlockSpec((1, 1, nb), lambda j: (jnp.maximum(j, 1) - 1, 0, 0)),
            pl.BlockSpec((batch, dim), lambda j: (0, 0)),
            pl.BlockSpec((batch, dim), lambda j: (0, 0)),
            pl.BlockSpec((batch, 2 * dim), lambda j: (0, 0)),
            pl.BlockSpec((2 * dim, 4 * dim), lambda j: (0, 0)),
            pl.BlockSpec((dim, 4 * dim), lambda j: (0, 0)),
            pl.BlockSpec((1, 4 * dim), lambda j: (0, 0)),
        ],
        out_specs=[
            pl.BlockSpec((batch, dim), lambda j: (0, 0)),
            pl.BlockSpec((batch, dim), lambda j: (0, 0)),
            pl.BlockSpec((1, 1, nb), lambda j: (jnp.maximum(j, 1) - 1, 0, 0)),
        ],
        out_shape=[
            jax.ShapeDtypeStruct((batch, dim), jnp.float32),
            jax.ShapeDtypeStruct((batch, dim), jnp.float32),
            jax.ShapeDtypeStruct((nblk, 1, nb), jnp.float32),
        ],
        compiler_params=pltpu.CompilerParams(
            dimension_semantics=("arbitrary",),
        ),
    )

    mesh = plsc.VectorSubcoreMesh(core_axis_name="c", subcore_axis_name="s",
                                  num_cores=1, num_subcores=_SUBCORES)
    sc_softmax = pl.kernel(
        functools.partial(_sc_softmax_body, chunk=chunk,
                          last_chunk=last_chunk, groups=groups, batch=batch),
        out_type=[jax.ShapeDtypeStruct((npd,), jnp.float32),
                  jax.ShapeDtypeStruct((batch,), jnp.float32)],
        mesh=mesh,
        scratch_types=[
            pltpu.VMEM((chunk,), jnp.float32),
            pltpu.VMEM((chunk,), jnp.int32),
            pltpu.VMEM((chunk,), jnp.float32),
            pltpu.VMEM((batch,), jnp.float32),
            pltpu.VMEM((batch,), jnp.float32),
            pltpu.VMEM((2 * _LANES * batch,), jnp.float32),
            pltpu.VMEM((2 * _LANES * batch,), jnp.float32),
            pltpu.VMEM((_SUBCORES, batch), jnp.float32),
            pltpu.VMEM_SHARED((_SUBCORES, batch), jnp.float32),
            pltpu.VMEM_SHARED((_SUBCORES, batch), jnp.float32),
        ],
        compiler_params=pltpu.CompilerParams(needs_layout_passes=False),
    )

    tc_ba = pl.pallas_call(
        functools.partial(_pool_lstm_scores_body, nblk=nblk, dim=dim),
        grid=(2 * nblk + 1,),
        in_specs=[
            pl.BlockSpec((nb, dim), lambda j: (
                jnp.where(j < nblk, j, jnp.maximum(j - nblk - 1, 0)), 0)),
            pl.BlockSpec((1, 1, nb), lambda j: (
                jnp.where(j < nblk, j, jnp.maximum(j - nblk - 1, 0)), 0, 0)),
            pl.BlockSpec((1, 1, nb), lambda j: (jnp.minimum(j, nblk - 1),
                                                0, 0)),
            pl.BlockSpec((1, batch), lambda j: (0, 0)),
            pl.BlockSpec((batch, dim), lambda j: (0, 0)),
            pl.BlockSpec((batch, dim), lambda j: (0, 0)),
            pl.BlockSpec((2 * dim, 4 * dim), lambda j: (0, 0)),
            pl.BlockSpec((dim, 4 * dim), lambda j: (0, 0)),
            pl.BlockSpec((1, 4 * dim), lambda j: (0, 0)),
        ],
        out_specs=[
            pl.BlockSpec((batch, dim), lambda j: (0, 0)),
            pl.BlockSpec((batch, dim), lambda j: (0, 0)),
            pl.BlockSpec((1, 1, nb), lambda j: (jnp.maximum(j - nblk - 1, 0),
                                                0, 0)),
        ],
        out_shape=[
            jax.ShapeDtypeStruct((batch, dim), jnp.float32),
            jax.ShapeDtypeStruct((batch, dim), jnp.float32),
            jax.ShapeDtypeStruct((nblk, 1, nb), jnp.float32),
        ],
        scratch_shapes=[pltpu.VMEM((batch, dim), jnp.float32)],
        compiler_params=pltpu.CompilerParams(
            dimension_semantics=("arbitrary",),
        ),
    )

    tc_pool = pl.pallas_call(
        functools.partial(_pool_body, nblk=nblk, dim=dim),
        grid=(nblk,),
        in_specs=[
            pl.BlockSpec((nb, dim), lambda j: (j, 0)),
            pl.BlockSpec((1, 1, nb), lambda j: (j, 0, 0)),
            pl.BlockSpec((1, 1, nb), lambda j: (j, 0, 0)),
            pl.BlockSpec((1, batch), lambda j: (0, 0)),
            pl.BlockSpec((batch, dim), lambda j: (0, 0)),
        ],
        out_specs=pl.BlockSpec((batch, 2 * dim), lambda j: (0, 0)),
        out_shape=jax.ShapeDtypeStruct((batch, 2 * dim), jnp.float32),
        scratch_shapes=[pltpu.VMEM((batch, dim), jnp.float32)],
        compiler_params=pltpu.CompilerParams(
            dimension_semantics=("arbitrary",),
        ),
    )

    xb = jnp.pad(x.astype(jnp.bfloat16), ((0, npd - n), (0, 0)))
    h = jnp.zeros((batch, dim), jnp.float32)
    c = jnp.zeros((batch, dim), jnp.float32)
    qs0 = jnp.zeros((batch, 2 * dim), jnp.float32)
    h, c, p = tc_scores(xb, n2g3, h, c, qs0, wih_t, whh_t, bias)
    for _ in range(_NUM_STEP - 1):
        e, norm = sc_softmax(p.reshape(-1), n2gp)
        h, c, p = tc_ba(xb, n2g3, e.reshape(nblk, 1, nb),
                        norm.reshape(1, batch), h, c, wih_t, whh_t, bias)
    e, norm = sc_softmax(p.reshape(-1), n2gp)
    return tc_pool(xb, n2g3, e.reshape(nblk, 1, nb), norm.reshape(1, batch),
                   h)


# speedup vs baseline: 1.1671x; 1.1494x over previous
import functools

import jax
import jax.numpy as jnp
from jax import lax
from jax.experimental import pallas as pl
from jax.experimental.pallas import tpu as pltpu
from jax.experimental.pallas import tpu_sc as plsc

_NUM_STEP = 3
_NEG = -1e30
_EPS = 1e-10
_BATCH = 128
_LANES = 16
_SUBCORES = 16


def _lstm_scores_body(x_ref, n2g_ref, h_in, c_in, qs_in, wih_ref, whh_ref,
                      bias_ref, h_out, c_out, p_out, xb_out, *, nblk, dim,
                      last_valid):
    j = pl.program_id(0)

    @pl.when(j == 0)
    def lstm_stage():
        gates = (
            lax.dot_general(qs_in[...], wih_ref[...], (((1,), (0,)), ((), ())),
                            preferred_element_type=jnp.float32)
            + lax.dot_general(h_in[...], whh_ref[...], (((1,), (0,)), ((), ())),
                              preferred_element_type=jnp.float32)
            + bias_ref[...]
        )
        i = jax.nn.sigmoid(gates[:, 0 * dim:1 * dim])
        f = jax.nn.sigmoid(gates[:, 1 * dim:2 * dim])
        g = jnp.tanh(gates[:, 2 * dim:3 * dim])
        o = jax.nn.sigmoid(gates[:, 3 * dim:4 * dim])
        c_new = f * c_in[...] + i * g
        c_out[...] = c_new
        h_out[...] = o * jnp.tanh(c_new)

    def _do_scores(x_bf):
        xb_out[...] = x_bf
        n2g = n2g_ref[0, 0, :]
        h = h_out[...].astype(jnp.bfloat16)
        scores_t = lax.dot_general(h, x_bf, (((1,), (1,)), ((), ())),
                                   preferred_element_type=jnp.float32)
        gid = lax.broadcasted_iota(jnp.int32, scores_t.shape, 0)
        member = gid == n2g[None, :]
        masked = jnp.where(member, scores_t, 0.0)
        ones = jnp.ones((1, masked.shape[0]), jnp.float32)
        p_out[...] = lax.dot_general(ones, masked, (((1,), (0,)), ((), ())),
                                     preferred_element_type=jnp.float32
                                     ).reshape(1, 1, -1)

    @pl.when((j > 0) & (j < nblk))
    def score_stage():
        _do_scores(x_ref[...].astype(jnp.bfloat16))

    @pl.when(j == nblk)
    def score_stage_tail():
        rid = lax.broadcasted_iota(jnp.int32, x_ref.shape, 0)
        x_bf = jnp.where(rid < last_valid, x_ref[...], 0.0
                         ).astype(jnp.bfloat16)
        _do_scores(x_bf)


def _sc_softmax_body(p_hbm, g_hbm, a_hbm, norm_hbm, p_v, g_v, e_v, m_v,
                     s_v, m2_v, s2_v, stage_v, sh_m, sh_s, *,
                     chunk, last_chunk, groups, batch):
    wid = lax.axis_index("s")
    base = wid * chunk
    lane_off = lax.iota(jnp.int32, _LANES) * batch
    tail = chunk - last_chunk

    @pl.when(wid < _SUBCORES - 1)
    def full_load():
        pltpu.sync_copy(p_hbm.at[pl.ds(base, chunk)], p_v)
        pltpu.sync_copy(g_hbm.at[pl.ds(base, chunk)], g_v)

    @pl.when(wid == _SUBCORES - 1)
    def tail_load():
        pltpu.sync_copy(p_hbm.at[pl.ds(base, last_chunk)],
                        p_v.at[pl.ds(0, last_chunk)])
        pltpu.sync_copy(g_hbm.at[pl.ds(base, last_chunk)],
                        g_v.at[pl.ds(0, last_chunk)])
        for i in range(tail // _LANES):
            sl = pl.ds(last_chunk + i * _LANES, _LANES)
            p_v[sl] = jnp.full((_LANES,), _NEG, jnp.float32)
            g_v[sl] = jnp.full((_LANES,), batch - 1, jnp.int32)

    for i in range(2 * batch):
        m2_v[pl.ds(i * _LANES, _LANES)] = jnp.full((_LANES,), _NEG,
                                                   jnp.float32)
        s2_v[pl.ds(i * _LANES, _LANES)] = jnp.zeros((_LANES,), jnp.float32)
    set1 = _LANES * batch

    def max_group(i, carry):
        sl0 = pl.ds((2 * i) * _LANES, _LANES)
        sl1 = pl.ds((2 * i + 1) * _LANES, _LANES)
        idx0 = lane_off + g_v[sl0]
        idx1 = set1 + lane_off + g_v[sl1]
        old0 = plsc.load_gather(m2_v, [idx0])
        old1 = plsc.load_gather(m2_v, [idx1])
        plsc.store_scatter(m2_v, [idx0], jnp.maximum(old0, p_v[sl0]))
        plsc.store_scatter(m2_v, [idx1], jnp.maximum(old1, p_v[sl1]))
        return carry

    lax.fori_loop(0, groups // 2, max_group, 0)

    for c in range(batch // _LANES):
        acc = jnp.full((_LANES,), _NEG, jnp.float32)
        for s in range(2 * _LANES):
            acc = jnp.maximum(acc, m2_v[pl.ds(s * batch + c * _LANES, _LANES)])
        m_v[pl.ds(c * _LANES, _LANES)] = acc
    pltpu.sync_copy(m_v, sh_m.at[wid])
    plsc.subcore_barrier()
    pltpu.sync_copy(sh_m, stage_v)
    for c in range(batch // _LANES):
        acc = jnp.full((_LANES,), _NEG, jnp.float32)
        for r in range(_SUBCORES):
            acc = jnp.maximum(acc, stage_v[r, pl.ds(c * _LANES, _LANES)])
        m_v[pl.ds(c * _LANES, _LANES)] = acc

    def sum_group(i, carry):
        sl0 = pl.ds((2 * i) * _LANES, _LANES)
        sl1 = pl.ds((2 * i + 1) * _LANES, _LANES)
        gv0 = g_v[sl0]
        gv1 = g_v[sl1]
        ev0 = jnp.exp(p_v[sl0] - plsc.load_gather(m_v, [gv0]))
        ev1 = jnp.exp(p_v[sl1] - plsc.load_gather(m_v, [gv1]))
        e_v[sl0] = ev0
        e_v[sl1] = ev1
        idx0 = lane_off + gv0
        idx1 = set1 + lane_off + gv1
        old0 = plsc.load_gather(s2_v, [idx0])
        old1 = plsc.load_gather(s2_v, [idx1])
        plsc.store_scatter(s2_v, [idx0], old0 + ev0)
        plsc.store_scatter(s2_v, [idx1], old1 + ev1)
        return carry

    lax.fori_loop(0, groups // 2, sum_group, 0)

    for c in range(batch // _LANES):
        acc = jnp.zeros((_LANES,), jnp.float32)
        for s in range(2 * _LANES):
            acc = acc + s2_v[pl.ds(s * batch + c * _LANES, _LANES)]
        s_v[pl.ds(c * _LANES, _LANES)] = acc
    pltpu.sync_copy(s_v, sh_s.at[wid])
    plsc.subcore_barrier()
    pltpu.sync_copy(sh_s, stage_v)
    for c in range(batch // _LANES):
        acc = jnp.zeros((_LANES,), jnp.float32)
        for r in range(_SUBCORES):
            acc = acc + stage_v[r, pl.ds(c * _LANES, _LANES)]
        s_v[pl.ds(c * _LANES, _LANES)] = acc + _EPS

    @pl.when(wid == 0)
    def norm_store():
        pltpu.sync_copy(s_v, norm_hbm)

    @pl.when(wid < _SUBCORES - 1)
    def full_store():
        pltpu.sync_copy(e_v, a_hbm.at[pl.ds(base, chunk)])

    @pl.when(wid == _SUBCORES - 1)
    def tail_store():
        pltpu.sync_copy(e_v.at[pl.ds(0, last_chunk)],
                        a_hbm.at[pl.ds(base, last_chunk)])


def _pool_lstm_scores_body(x_ref, n2g_ref, a_ref, norm_ref, h_in, c_in,
                           wih_ref, whh_ref, bias_ref, h_out, c_out, p_out,
                           o_acc, *, nblk, dim):
    j = pl.program_id(0)

    @pl.when(j == 0)
    def init():
        o_acc[...] = jnp.zeros_like(o_acc)

    @pl.when(j < nblk)
    def pool_stage():
        x_blk = x_ref[...]
        n2g = n2g_ref[0, 0, :]
        a_blk = a_ref[0, :, :].astype(jnp.bfloat16)
        gid = lax.broadcasted_iota(jnp.int16,
                                   (o_acc.shape[0], x_blk.shape[0]), 0)
        n2gb = n2g.astype(jnp.int16)
        amask = jnp.where(gid == n2gb[None, :], a_blk, jnp.bfloat16(0))
        o_acc[...] += lax.dot_general(amask, x_blk, (((1,), (0,)), ((), ())),
                                      preferred_element_type=jnp.float32)

    @pl.when(j == nblk)
    def lstm_stage():
        pooled = o_acc[...] / norm_ref[...].reshape(o_acc.shape[0], 1)
        gates = (
            lax.dot_general(h_in[...], wih_ref[0:dim, :],
                            (((1,), (0,)), ((), ())),
                            preferred_element_type=jnp.float32)
            + lax.dot_general(pooled, wih_ref[dim:2 * dim, :],
                              (((1,), (0,)), ((), ())),
                              preferred_element_type=jnp.float32)
            + lax.dot_general(h_in[...], whh_ref[...],
                              (((1,), (0,)), ((), ())),
                              preferred_element_type=jnp.float32)
            + bias_ref[...]
        )
        i = jax.nn.sigmoid(gates[:, 0 * dim:1 * dim])
        f = jax.nn.sigmoid(gates[:, 1 * dim:2 * dim])
        g = jnp.tanh(gates[:, 2 * dim:3 * dim])
        o = jax.nn.sigmoid(gates[:, 3 * dim:4 * dim])
        c_new = f * c_in[...] + i * g
        c_out[...] = c_new
        h_out[...] = o * jnp.tanh(c_new)

    @pl.when(j > nblk)
    def score_stage():
        x_blk = x_ref[...]
        n2g = n2g_ref[0, 0, :]
        h = h_out[...].astype(jnp.bfloat16)
        scores_t = lax.dot_general(h, x_blk, (((1,), (1,)), ((), ())),
                                   preferred_element_type=jnp.float32)
        gid = lax.broadcasted_iota(jnp.int32, scores_t.shape, 0)
        masked = jnp.where(gid == n2g[None, :], scores_t, 0.0)
        ones = jnp.ones((1, masked.shape[0]), jnp.float32)
        p_out[...] = lax.dot_general(ones, masked, (((1,), (0,)), ((), ())),
                                     preferred_element_type=jnp.float32
                                     ).reshape(1, 1, -1)


def _pool_body(x_ref, n2g_ref, a_ref, norm_ref, h_in, qs_out, o_acc, *,
               nblk, dim):
    j = pl.program_id(0)

    @pl.when(j == 0)
    def init():
        o_acc[...] = jnp.zeros_like(o_acc)

    x_blk = x_ref[...]
    n2g = n2g_ref[0, 0, :]
    a_blk = a_ref[0, :, :].astype(jnp.bfloat16)
    gid = lax.broadcasted_iota(jnp.int16,
                               (o_acc.shape[0], x_blk.shape[0]), 0)
    amask = jnp.where(gid == n2g.astype(jnp.int16)[None, :], a_blk,
                      jnp.bfloat16(0))
    o_acc[...] += lax.dot_general(amask, x_blk, (((1,), (0,)), ((), ())),
                                  preferred_element_type=jnp.float32)

    @pl.when(j == nblk - 1)
    def finish():
        qs_out[:, 0:dim] = h_in[...]
        qs_out[:, dim:2 * dim] = (o_acc[...]
                                  / norm_ref[...].reshape(o_acc.shape[0], 1))


def kernel(input, node2graph, batch_size, W_ih, W_hh, b_ih, b_hh):
    x = input
    n, dim = x.shape
    batch = _BATCH

    nb = 12800
    npd = -(-n // nb) * nb
    nblk = npd // nb

    chunk = -(-n // (_SUBCORES * _LANES)) * _LANES
    last_chunk = n - (_SUBCORES - 1) * chunk
    assert 0 < last_chunk <= chunk and last_chunk % _LANES == 0
    assert chunk % _LANES == 0 and (chunk // _LANES) % 2 == 0
    groups = chunk // _LANES

    n2gp = jnp.pad(node2graph.astype(jnp.int32), (0, npd - n),
                   constant_values=-1)
    n2g3 = n2gp.reshape(nblk, 1, nb)
    wih_t = W_ih.T
    whh_t = W_hh.T
    bias = (b_ih + b_hh).reshape(1, 4 * dim)

    last_valid = n - (nblk - 1) * nb
    tc_scores = pl.pallas_call(
        functools.partial(_lstm_scores_body, nblk=nblk, dim=dim,
                          last_valid=last_valid),
        grid=(nblk + 1,),
        in_specs=[
            pl.BlockSpec((nb, dim), lambda j: (jnp.maximum(j, 1) - 1, 0)),
            pl.BlockSpec((1, 1, nb), lambda j: (jnp.maximum(j, 1) - 1, 0, 0)),
            pl.BlockSpec((batch, dim), lambda j: (0, 0)),
            pl.BlockSpec((batch, dim), lambda j: (0, 0)),
            pl.BlockSpec((batch, 2 * dim), lambda j: (0, 0)),
            pl.BlockSpec((2 * dim, 4 * dim), lambda j: (0, 0)),
            pl.BlockSpec((dim, 4 * dim), lambda j: (0, 0)),
            pl.BlockSpec((1, 4 * dim), lambda j: (0, 0)),
        ],
        out_specs=[
            pl.BlockSpec((batch, dim), lambda j: (0, 0)),
            pl.BlockSpec((batch, dim), lambda j: (0, 0)),
            pl.BlockSpec((1, 1, nb), lambda j: (jnp.maximum(j, 1) - 1, 0, 0)),
            pl.BlockSpec((nb, dim), lambda j: (jnp.maximum(j, 1) - 1, 0)),
        ],
        out_shape=[
            jax.ShapeDtypeStruct((batch, dim), jnp.float32),
            jax.ShapeDtypeStruct((batch, dim), jnp.float32),
            jax.ShapeDtypeStruct((nblk, 1, nb), jnp.float32),
            jax.ShapeDtypeStruct((npd, dim), jnp.bfloat16),
        ],
        compiler_params=pltpu.CompilerParams(
            dimension_semantics=("arbitrary",),
        ),
    )

    mesh = plsc.VectorSubcoreMesh(core_axis_name="c", subcore_axis_name="s",
                                  num_cores=1, num_subcores=_SUBCORES)
    sc_softmax = pl.kernel(
        functools.partial(_sc_softmax_body, chunk=chunk,
                          last_chunk=last_chunk, groups=groups, batch=batch),
        out_type=[jax.ShapeDtypeStruct((npd,), jnp.float32),
                  jax.ShapeDtypeStruct((batch,), jnp.float32)],
        mesh=mesh,
        scratch_types=[
            pltpu.VMEM((chunk,), jnp.float32),
            pltpu.VMEM((chunk,), jnp.int32),
            pltpu.VMEM((chunk,), jnp.float32),
            pltpu.VMEM((batch,), jnp.float32),
            pltpu.VMEM((batch,), jnp.float32),
            pltpu.VMEM((2 * _LANES * batch,), jnp.float32),
            pltpu.VMEM((2 * _LANES * batch,), jnp.float32),
            pltpu.VMEM((_SUBCORES, batch), jnp.float32),
            pltpu.VMEM_SHARED((_SUBCORES, batch), jnp.float32),
            pltpu.VMEM_SHARED((_SUBCORES, batch), jnp.float32),
        ],
        compiler_params=pltpu.CompilerParams(needs_layout_passes=False),
    )

    tc_ba = pl.pallas_call(
        functools.partial(_pool_lstm_scores_body, nblk=nblk, dim=dim),
        grid=(2 * nblk + 1,),
        in_specs=[
            pl.BlockSpec((nb, dim), lambda j: (
                jnp.where(j < nblk, j, jnp.maximum(j - nblk - 1, 0)), 0)),
            pl.BlockSpec((1, 1, nb), lambda j: (
                jnp.where(j < nblk, j, jnp.maximum(j - nblk - 1, 0)), 0, 0)),
            pl.BlockSpec((1, 1, nb), lambda j: (jnp.minimum(j, nblk - 1),
                                                0, 0)),
            pl.BlockSpec((1, batch), lambda j: (0, 0)),
            pl.BlockSpec((batch, dim), lambda j: (0, 0)),
            pl.BlockSpec((batch, dim), lambda j: (0, 0)),
            pl.BlockSpec((2 * dim, 4 * dim), lambda j: (0, 0)),
            pl.BlockSpec((dim, 4 * dim), lambda j: (0, 0)),
            pl.BlockSpec((1, 4 * dim), lambda j: (0, 0)),
        ],
        out_specs=[
            pl.BlockSpec((batch, dim), lambda j: (0, 0)),
            pl.BlockSpec((batch, dim), lambda j: (0, 0)),
            pl.BlockSpec((1, 1, nb), lambda j: (jnp.maximum(j - nblk - 1, 0),
                                                0, 0)),
        ],
        out_shape=[
            jax.ShapeDtypeStruct((batch, dim), jnp.float32),
            jax.ShapeDtypeStruct((batch, dim), jnp.float32),
            jax.ShapeDtypeStruct((nblk, 1, nb), jnp.float32),
        ],
        scratch_shapes=[pltpu.VMEM((batch, dim), jnp.float32)],
        compiler_params=pltpu.CompilerParams(
            dimension_semantics=("arbitrary",),
        ),
    )

    tc_pool = pl.pallas_call(
        functools.partial(_pool_body, nblk=nblk, dim=dim),
        grid=(nblk,),
        in_specs=[
            pl.BlockSpec((nb, dim), lambda j: (j, 0)),
            pl.BlockSpec((1, 1, nb), lambda j: (j, 0, 0)),
            pl.BlockSpec((1, 1, nb), lambda j: (j, 0, 0)),
            pl.BlockSpec((1, batch), lambda j: (0, 0)),
            pl.BlockSpec((batch, dim), lambda j: (0, 0)),
        ],
        out_specs=pl.BlockSpec((batch, 2 * dim), lambda j: (0, 0)),
        out_shape=jax.ShapeDtypeStruct((batch, 2 * dim), jnp.float32),
        scratch_shapes=[pltpu.VMEM((batch, dim), jnp.float32)],
        compiler_params=pltpu.CompilerParams(
            dimension_semantics=("arbitrary",),
        ),
    )

    h = jnp.zeros((batch, dim), jnp.float32)
    c = jnp.zeros((batch, dim), jnp.float32)
    qs0 = jnp.zeros((batch, 2 * dim), jnp.float32)
    h, c, p, xb = tc_scores(x, n2g3, h, c, qs0, wih_t, whh_t, bias)
    for _ in range(_NUM_STEP - 1):
        e, norm = sc_softmax(p.reshape(-1), n2gp)
        h, c, p = tc_ba(xb, n2g3, e.reshape(nblk, 1, nb),
                        norm.reshape(1, batch), h, c, wih_t, whh_t, bias)
    e, norm = sc_softmax(p.reshape(-1), n2gp)
    return tc_pool(xb, n2g3, e.reshape(nblk, 1, nb), norm.reshape(1, batch),
                   h)
